# baseline (speedup 1.0000x reference)
import functools

import jax
import jax.numpy as jnp
from jax import lax
from jax.experimental import pallas as pl
from jax.experimental.pallas import tpu as pltpu
from jax.experimental.pallas import tpu_sc as plsc

NS = 16
NC = 2
L = 16
DH = 64


def _tc_tables_body(x_ref, w_ref, a1_ref, a2_ref, ai2_ref, jh_ref):
    h = jnp.dot(x_ref[...], w_ref[...], preferred_element_type=jnp.float32)
    ai = jnp.dot(h, a1_ref[...], preferred_element_type=jnp.float32)
    aj = jnp.dot(h, a2_ref[...], preferred_element_type=jnp.float32)
    ai2_ref[0] = ai[:, :DH]
    ai2_ref[1] = ai[:, DH:]
    jh_ref[0] = jnp.concatenate([aj[:, :DH], h[:, :DH]], axis=1)
    jh_ref[1] = jnp.concatenate([aj[:, DH:], h[:, DH:]], axis=1)


def _tc_tables(x, w, a1, a2):
    n, d = x.shape
    bn = 1000
    assert n % bn == 0
    return pl.pallas_call(
        _tc_tables_body,
        grid=(n // bn,),
        in_specs=[
            pl.BlockSpec((bn, d), lambda i: (i, 0)),
            pl.BlockSpec((d, d), lambda i: (0, 0)),
            pl.BlockSpec((d, d), lambda i: (0, 0)),
            pl.BlockSpec((d, d), lambda i: (0, 0)),
        ],
        out_specs=[
            pl.BlockSpec((NC, bn, DH), lambda i: (0, i, 0)),
            pl.BlockSpec((NC, bn, 2 * DH), lambda i: (0, i, 0)),
        ],
        out_shape=[
            jax.ShapeDtypeStruct((NC, n, DH), jnp.float32),
            jax.ShapeDtypeStruct((NC, n, 2 * DH), jnp.float32),
        ],
    )(x, w, a1, a2)


def _make_sc_edge_kernel(n, e, npad):
    K = 80
    NSUB = 10
    SCH = NSUB * K
    EC = e // NS
    NSUPER = EC // SCH
    RCH = 8
    RPT = npad // NS
    assert EC * NS == e and NSUPER * SCH == EC and NSUB % 2 == 0
    assert RPT % RCH == 0 and RPT % 8 == 0 and npad >= n and n % RCH == 0

    mesh = plsc.VectorSubcoreMesh(core_axis_name="c", subcore_axis_name="s")

    @functools.partial(
        pl.kernel,
        out_type=jax.ShapeDtypeStruct((n, NC * DH), jnp.float32),
        mesh=mesh,
        scratch_types=[
            pltpu.VMEM((2, NSUB, K), jnp.int32),
            pltpu.VMEM((2, NSUB, K), jnp.int32),
            pltpu.VMEM((2, NSUB, K), jnp.int32),
            pltpu.VMEM((2, K, DH), jnp.float32),
            pltpu.VMEM((2, K, 2 * DH), jnp.float32),
            pltpu.VMEM((K, 2 * DH), jnp.float32),
            pltpu.VMEM_SHARED((npad, 2 * DH), jnp.float32),
            pltpu.VMEM((RCH, 2 * DH), jnp.float32),
            pltpu.VMEM((RCH, DH), jnp.float32),
            pltpu.VMEM((DH,), jnp.float32),
            pltpu.SemaphoreType.DMA,
            pltpu.SemaphoreType.DMA,
            pltpu.SemaphoreType.DMA,
            pltpu.SemaphoreType.DMA,
            pltpu.SemaphoreType.DMA,
            pltpu.SemaphoreType.DMA,
        ],
        compiler_params=pltpu.CompilerParams(use_tc_tiling_on_sc=False),
    )
    def sc_edge(src_hbm, dst_hbm, ai_hbm, jh_hbm, bias_hbm, out_hbm,
                srcv, srcg, dstg, abuf, jhbuf, vpbuf, numden,
                ndbuf, obuf, biasv, sema0, sema1, semj0, semj1, semi, semv):
        c = lax.axis_index("c")
        s = lax.axis_index("s")
        cn = c * n
        sem_a = (sema0, sema1)
        sem_j = (semj0, semj1)

        zero = jnp.zeros((L,), jnp.float32)

        def zrow(r, _):
            for g in range(2 * DH // L):
                ndbuf[r, pl.ds(g * L, L)] = zero
            return 0

        lax.fori_loop(0, RCH, zrow, 0)

        def zchunk(jr, _):
            pltpu.sync_copy(ndbuf, numden.at[pl.ds(s * RPT + jr * RCH, RCH)])
            return 0

        lax.fori_loop(0, RPT // RCH, zchunk, 0)
        plsc.subcore_barrier()

        ebase = s * EC

        def fire_idx(sc_i, ib):
            base = ebase + sc_i * SCH
            for i in range(NSUB):
                pltpu.async_copy(src_hbm.at[pl.ds(base + i * K, K)],
                                 srcv.at[ib, i], semi)
                pltpu.async_copy(dst_hbm.at[pl.ds(base + i * K, K)],
                                 dstg.at[ib, i], semi)

        def wait_idx(sc_i, ib):
            base = ebase + sc_i * SCH
            for i in range(NSUB):
                pltpu.make_async_copy(src_hbm.at[pl.ds(base + i * K, K)],
                                      srcv.at[ib, i], semi).wait()
                pltpu.make_async_copy(dst_hbm.at[pl.ds(base + i * K, K)],
                                      dstg.at[ib, i], semi).wait()

        H = K // 2

        def fire(ib, k, buf):
            for q in range(2):
                pltpu.async_copy(ai_hbm.at[srcg.at[ib, k, pl.ds(q * H, H)]],
                                 abuf.at[buf, pl.ds(q * H, H)], sem_a[buf])
                pltpu.async_copy(jh_hbm.at[dstg.at[ib, k, pl.ds(q * H, H)]],
                                 jhbuf.at[buf, pl.ds(q * H, H)], sem_j[buf])

        def wait_gathers(ib, k, buf):
            for q in range(2):
                pltpu.make_async_copy(
                    ai_hbm.at[srcg.at[ib, k, pl.ds(q * H, H)]],
                    abuf.at[buf, pl.ds(q * H, H)], sem_a[buf]).wait()
                pltpu.make_async_copy(
                    jh_hbm.at[dstg.at[ib, k, pl.ds(q * H, H)]],
                    jhbuf.at[buf, pl.ds(q * H, H)], sem_j[buf]).wait()

        def wait_scatter():
            pltpu.make_async_copy(vpbuf, numden.at[srcv.at[0, 0]],
                                  semv).wait()

        def compute(buf):
            @plsc.parallel_loop(0, K, 1, unroll=8)
            def edge(ei):
                for g in range(DH // L):
                    a = abuf[buf, ei, pl.ds(g * L, L)]
                    b = jhbuf[buf, ei, pl.ds(g * L, L)]
                    hh = jhbuf[buf, ei, pl.ds(DH + g * L, L)]
                    lg = a + b
                    lg = jnp.maximum(lg, 0.2 * lg)
                    p = jnp.exp(lg)
                    vpbuf[ei, pl.ds(g * L, L)] = p * hh
                    vpbuf[ei, pl.ds(DH + g * L, L)] = p

        def fire_scatter(ib, k):
            pltpu.async_copy(vpbuf, numden.at[srcv.at[ib, k]], semv, add=True)

        def addcn(ib):
            def body(i, _):
                for q in range(K // L):
                    sl = pl.ds(q * L, L)
                    srcg[ib, i, sl] = srcv[ib, i, sl] + cn
                    dstg[ib, i, sl] = dstg[ib, i, sl] + cn
                return 0

            lax.fori_loop(0, NSUB, body, 0)

        fire_idx(0, 0)

        def superchunk(sc_i, _):
            ib = lax.rem(sc_i, 2)
            wait_idx(sc_i, ib)
            addcn(ib)

            @pl.when(sc_i + 1 < NSUPER)
            def _():
                fire_idx(sc_i + 1, 1 - ib)

            fire(ib, 0, 0)

            def pipe(k2, _):
                k = 2 * k2
                fire(ib, k + 1, 1)
                wait_gathers(ib, k, 0)

                @pl.when((k2 > 0) | (sc_i > 0))
                def _():
                    wait_scatter()

                compute(0)
                fire_scatter(ib, k)

                @pl.when(k + 2 < NSUB)
                def _():
                    fire(ib, k + 2, 0)

                wait_gathers(ib, k + 1, 1)
                wait_scatter()
                compute(1)
                fire_scatter(ib, k + 1)
                return 0

            lax.fori_loop(0, NSUB // 2, pipe, 0)
            return 0

        lax.fori_loop(0, NSUPER, superchunk, 0)
        wait_scatter()
        plsc.subcore_barrier()

        pltpu.sync_copy(bias_hbm.at[pl.ds(c * DH, DH)], biasv)

        def rchunk(jr, _):
            r0 = s * RPT + jr * RCH
            pltpu.sync_copy(numden.at[pl.ds(r0, RCH)], ndbuf)

            def row(r, _):
                for g in range(DH // L):
                    nm = ndbuf[r, pl.ds(g * L, L)]
                    dn = ndbuf[r, pl.ds(DH + g * L, L)]
                    bv = biasv[pl.ds(g * L, L)]
                    obuf[r, pl.ds(g * L, L)] = nm / (dn + 1e-16) + bv
                return 0

            lax.fori_loop(0, RCH, row, 0)

            @pl.when(r0 < n)
            def _():
                pltpu.sync_copy(obuf,
                                out_hbm.at[pl.ds(r0, RCH),
                                           pl.ds(c * DH, DH)])

            return 0

        lax.fori_loop(0, RPT // RCH, rchunk, 0)

    return sc_edge


def kernel(nodes_ft, adj_list, weight, bias, att_layer_1, att_layer_2):
    n, d = nodes_ft.shape
    e = adj_list.shape[1]
    npad = ((n + 127) // 128) * 128
    ai2, jh = _tc_tables(nodes_ft, weight, att_layer_1, att_layer_2)
    sc_edge = _make_sc_edge_kernel(n, e, npad)
    return sc_edge(
        adj_list[0],
        adj_list[1],
        ai2.reshape(NC * n, DH),
        jh.reshape(NC * n, 2 * DH),
        bias,
    )


# speedup vs baseline: 1.0508x; 1.0508x over previous
import functools

import jax
import jax.numpy as jnp
from jax import lax
from jax.experimental import pallas as pl
from jax.experimental.pallas import tpu as pltpu
from jax.experimental.pallas import tpu_sc as plsc

NS = 16
NC = 2
L = 16
DH = 64


def _tc_tables_body(x_ref, w_ref, a1_ref, a2_ref, ai2_ref, jh_ref):
    h = jnp.dot(x_ref[...], w_ref[...], preferred_element_type=jnp.float32)
    ai = jnp.dot(h, a1_ref[...], preferred_element_type=jnp.float32)
    aj = jnp.dot(h, a2_ref[...], preferred_element_type=jnp.float32)
    ai2_ref[0] = ai[:, :DH]
    ai2_ref[1] = ai[:, DH:]
    jh_ref[0] = jnp.concatenate([aj[:, :DH], h[:, :DH]], axis=1)
    jh_ref[1] = jnp.concatenate([aj[:, DH:], h[:, DH:]], axis=1)


def _tc_tables(x, w, a1, a2):
    n, d = x.shape
    bn = 1000
    assert n % bn == 0
    return pl.pallas_call(
        _tc_tables_body,
        grid=(n // bn,),
        in_specs=[
            pl.BlockSpec((bn, d), lambda i: (i, 0)),
            pl.BlockSpec((d, d), lambda i: (0, 0)),
            pl.BlockSpec((d, d), lambda i: (0, 0)),
            pl.BlockSpec((d, d), lambda i: (0, 0)),
        ],
        out_specs=[
            pl.BlockSpec((NC, bn, DH), lambda i: (0, i, 0)),
            pl.BlockSpec((NC, bn, 2 * DH), lambda i: (0, i, 0)),
        ],
        out_shape=[
            jax.ShapeDtypeStruct((NC, n, DH), jnp.float32),
            jax.ShapeDtypeStruct((NC, n, 2 * DH), jnp.float32),
        ],
    )(x, w, a1, a2)


def _make_sc_edge_kernel(n, e, npad):
    K = 80
    NSUB = 10
    SCH = NSUB * K
    EC = e // NS
    NSUPER = EC // SCH
    RCH = 8
    RPT = npad // NS
    assert EC * NS == e and NSUPER * SCH == EC and NSUB % 2 == 0
    assert RPT % RCH == 0 and RPT % 8 == 0 and npad >= n and n % RCH == 0

    mesh = plsc.VectorSubcoreMesh(core_axis_name="c", subcore_axis_name="s")

    @functools.partial(
        pl.kernel,
        out_type=jax.ShapeDtypeStruct((npad, NC * DH), jnp.float32),
        mesh=mesh,
        scratch_types=[
            pltpu.VMEM((2, NSUB, K), jnp.int32),
            pltpu.VMEM((2, NSUB, K), jnp.int32),
            pltpu.VMEM((2, NSUB, K), jnp.int32),
            pltpu.VMEM((2, K, DH), jnp.float32),
            pltpu.VMEM((2, K, 2 * DH), jnp.float32),
            pltpu.VMEM((K, 2 * DH), jnp.float32),
            pltpu.VMEM_SHARED((npad, 2 * DH), jnp.float32),
            pltpu.VMEM((DH,), jnp.float32),
            pltpu.SemaphoreType.DMA,
            pltpu.SemaphoreType.DMA,
            pltpu.SemaphoreType.DMA,
            pltpu.SemaphoreType.DMA,
            pltpu.SemaphoreType.DMA,
            pltpu.SemaphoreType.DMA,
        ],
        compiler_params=pltpu.CompilerParams(use_tc_tiling_on_sc=False),
    )
    def sc_edge(src_hbm, dst_hbm, ai_hbm, jh_hbm, bias_hbm, out_hbm,
                srcv, srcg, dstg, abuf, jhbuf, vpbuf, numden,
                biasv, sema0, sema1, semj0, semj1, semi, semv):
        c = lax.axis_index("c")
        s = lax.axis_index("s")
        cn = c * n
        sem_a = (sema0, sema1)
        sem_j = (semj0, semj1)

        zero = jnp.zeros((L,), jnp.float32)
        zsrc = jhbuf.at[0, pl.ds(0, RCH)]

        def zrow(r, _):
            for g in range(2 * DH // L):
                jhbuf[0, r, pl.ds(g * L, L)] = zero
            return 0

        lax.fori_loop(0, RCH, zrow, 0)

        def zfire(jr, _):
            pltpu.async_copy(zsrc, numden.at[pl.ds(s * RPT + jr * RCH, RCH)],
                             semi)
            return 0

        lax.fori_loop(0, RPT // RCH, zfire, 0)

        def zwait(jr, _):
            pltpu.make_async_copy(
                zsrc, numden.at[pl.ds(s * RPT + jr * RCH, RCH)], semi).wait()
            return 0

        lax.fori_loop(0, RPT // RCH, zwait, 0)
        plsc.subcore_barrier()

        ebase = s * EC

        def fire_idx(sc_i, ib):
            base = ebase + sc_i * SCH
            for i in range(NSUB):
                pltpu.async_copy(src_hbm.at[pl.ds(base + i * K, K)],
                                 srcv.at[ib, i], semi)
                pltpu.async_copy(dst_hbm.at[pl.ds(base + i * K, K)],
                                 dstg.at[ib, i], semi)

        def wait_idx(sc_i, ib):
            base = ebase + sc_i * SCH
            for i in range(NSUB):
                pltpu.make_async_copy(src_hbm.at[pl.ds(base + i * K, K)],
                                      srcv.at[ib, i], semi).wait()
                pltpu.make_async_copy(dst_hbm.at[pl.ds(base + i * K, K)],
                                      dstg.at[ib, i], semi).wait()

        H = K // 2

        def fire(ib, k, buf):
            for q in range(2):
                pltpu.async_copy(ai_hbm.at[srcg.at[ib, k, pl.ds(q * H, H)]],
                                 abuf.at[buf, pl.ds(q * H, H)], sem_a[buf])
                pltpu.async_copy(jh_hbm.at[dstg.at[ib, k, pl.ds(q * H, H)]],
                                 jhbuf.at[buf, pl.ds(q * H, H)], sem_j[buf])

        def wait_gathers(ib, k, buf):
            for q in range(2):
                pltpu.make_async_copy(
                    ai_hbm.at[srcg.at[ib, k, pl.ds(q * H, H)]],
                    abuf.at[buf, pl.ds(q * H, H)], sem_a[buf]).wait()
                pltpu.make_async_copy(
                    jh_hbm.at[dstg.at[ib, k, pl.ds(q * H, H)]],
                    jhbuf.at[buf, pl.ds(q * H, H)], sem_j[buf]).wait()

        def wait_scatter():
            pltpu.make_async_copy(vpbuf, numden.at[srcv.at[0, 0]],
                                  semv).wait()

        def compute(buf):
            @plsc.parallel_loop(0, K, 1, unroll=8)
            def edge(ei):
                for g in range(DH // L):
                    a = abuf[buf, ei, pl.ds(g * L, L)]
                    b = jhbuf[buf, ei, pl.ds(g * L, L)]
                    hh = jhbuf[buf, ei, pl.ds(DH + g * L, L)]
                    lg = a + b
                    lg = jnp.maximum(lg, 0.2 * lg)
                    p = jnp.exp(lg)
                    vpbuf[ei, pl.ds(g * L, L)] = p * hh
                    vpbuf[ei, pl.ds(DH + g * L, L)] = p

        def fire_scatter(ib, k):
            pltpu.async_copy(vpbuf, numden.at[srcv.at[ib, k]], semv, add=True)

        def addcn(ib):
            def body(i, _):
                for q in range(K // L):
                    sl = pl.ds(q * L, L)
                    srcg[ib, i, sl] = srcv[ib, i, sl] + cn
                    dstg[ib, i, sl] = dstg[ib, i, sl] + cn
                return 0

            lax.fori_loop(0, NSUB, body, 0)

        fire_idx(0, 0)

        def superchunk(sc_i, _):
            ib = lax.rem(sc_i, 2)
            wait_idx(sc_i, ib)
            addcn(ib)

            @pl.when(sc_i + 1 < NSUPER)
            def _():
                fire_idx(sc_i + 1, 1 - ib)

            fire(ib, 0, 0)

            def pipe(k2, _):
                k = 2 * k2
                fire(ib, k + 1, 1)
                wait_gathers(ib, k, 0)

                @pl.when((k2 > 0) | (sc_i > 0))
                def _():
                    wait_scatter()

                compute(0)
                fire_scatter(ib, k)

                @pl.when(k + 2 < NSUB)
                def _():
                    fire(ib, k + 2, 0)

                wait_gathers(ib, k + 1, 1)
                wait_scatter()
                compute(1)
                fire_scatter(ib, k + 1)
                return 0

            lax.fori_loop(0, NSUB // 2, pipe, 0)
            return 0

        lax.fori_loop(0, NSUPER, superchunk, 0)
        wait_scatter()
        plsc.subcore_barrier()

        pltpu.sync_copy(bias_hbm.at[pl.ds(c * DH, DH)], biasv)
        NCHE = RPT // RCH

        def nd_dst(b):
            return jhbuf.at[b, pl.ds(0, RCH)]

        def o_src(b):
            return abuf.at[b, pl.ds(0, RCH)]

        def fire_nd(jr, b):
            pltpu.async_copy(numden.at[pl.ds(s * RPT + jr * RCH, RCH)],
                             nd_dst(b), sem_a[b])

        def wait_nd(jr, b):
            pltpu.make_async_copy(numden.at[pl.ds(s * RPT + jr * RCH, RCH)],
                                  nd_dst(b), sem_a[b]).wait()

        def fire_out(jr, b):
            pltpu.async_copy(o_src(b),
                             out_hbm.at[pl.ds(s * RPT + jr * RCH, RCH),
                                        pl.ds(c * DH, DH)], sem_j[b])

        def wait_out(jr, b):
            pltpu.make_async_copy(o_src(b),
                                  out_hbm.at[pl.ds(s * RPT + jr * RCH, RCH),
                                             pl.ds(c * DH, DH)],
                                  sem_j[b]).wait()

        def ecompute(b):
            @plsc.parallel_loop(0, RCH, 1, unroll=4)
            def row(r):
                for g in range(DH // L):
                    nm = jhbuf[b, r, pl.ds(g * L, L)]
                    dn = jhbuf[b, r, pl.ds(DH + g * L, L)]
                    bv = biasv[pl.ds(g * L, L)]
                    abuf[b, r, pl.ds(g * L, L)] = nm / (dn + 1e-16) + bv

        fire_nd(0, 0)

        def echunk(jr2, _):
            jr = 2 * jr2

            @pl.when(jr + 1 < NCHE)
            def _():
                fire_nd(jr + 1, 1)

            wait_nd(jr, 0)

            @pl.when(jr2 > 0)
            def _():
                wait_out(jr, 0)

            ecompute(0)
            fire_out(jr, 0)

            @pl.when(jr + 1 < NCHE)
            def _():
                @pl.when(jr + 2 < NCHE)
                def _():
                    fire_nd(jr + 2, 0)

                wait_nd(jr + 1, 1)

                @pl.when(jr2 > 0)
                def _():
                    wait_out(jr + 1, 1)

                ecompute(1)
                fire_out(jr + 1, 1)

            return 0

        lax.fori_loop(0, (NCHE + 1) // 2, echunk, 0)
        wait_out(0, 0)

        @pl.when(NCHE > 1)
        def _():
            wait_out(0, 1)

    return sc_edge


def kernel(nodes_ft, adj_list, weight, bias, att_layer_1, att_layer_2):
    n, d = nodes_ft.shape
    e = adj_list.shape[1]
    npad = ((n + 127) // 128) * 128
    ai2, jh = _tc_tables(nodes_ft, weight, att_layer_1, att_layer_2)
    sc_edge = _make_sc_edge_kernel(n, e, npad)
    out = sc_edge(
        adj_list[0],
        adj_list[1],
        ai2.reshape(NC * n, DH),
        jh.reshape(NC * n, 2 * DH),
        bias,
    )
    return out[:n]


# speedup vs baseline: 1.0510x; 1.0003x over previous
import functools

import jax
import jax.numpy as jnp
from jax import lax
from jax.experimental import pallas as pl
from jax.experimental.pallas import tpu as pltpu
from jax.experimental.pallas import tpu_sc as plsc

NS = 16
NC = 2
L = 16
DH = 64


def _tc_tables_body(x_ref, w_ref, a1_ref, a2_ref, ai2_ref, jh_ref):
    h = jnp.dot(x_ref[...], w_ref[...], preferred_element_type=jnp.float32)
    ai = jnp.dot(h, a1_ref[...], preferred_element_type=jnp.float32)
    aj = jnp.dot(h, a2_ref[...], preferred_element_type=jnp.float32)
    ai2_ref[0] = ai[:, :DH]
    ai2_ref[1] = ai[:, DH:]
    jh_ref[0] = jnp.concatenate([aj[:, :DH], h[:, :DH]], axis=1)
    jh_ref[1] = jnp.concatenate([aj[:, DH:], h[:, DH:]], axis=1)


def _tc_tables(x, w, a1, a2):
    n, d = x.shape
    bn = 1000
    assert n % bn == 0
    return pl.pallas_call(
        _tc_tables_body,
        grid=(n // bn,),
        in_specs=[
            pl.BlockSpec((bn, d), lambda i: (i, 0)),
            pl.BlockSpec((d, d), lambda i: (0, 0)),
            pl.BlockSpec((d, d), lambda i: (0, 0)),
            pl.BlockSpec((d, d), lambda i: (0, 0)),
        ],
        out_specs=[
            pl.BlockSpec((NC, bn, DH), lambda i: (0, i, 0)),
            pl.BlockSpec((NC, bn, 2 * DH), lambda i: (0, i, 0)),
        ],
        out_shape=[
            jax.ShapeDtypeStruct((NC, n, DH), jnp.float32),
            jax.ShapeDtypeStruct((NC, n, 2 * DH), jnp.float32),
        ],
    )(x, w, a1, a2)


def _make_sc_edge_kernel(n, e, npad):
    K = 80
    NSUB = 10
    SCH = NSUB * K
    EC = e // NS
    NSUPER = EC // SCH
    RCH = 8
    RPT = npad // NS
    assert EC * NS == e and NSUPER * SCH == EC and NSUB % 2 == 0
    assert RPT % RCH == 0 and RPT % 8 == 0 and npad >= n and n % RCH == 0

    mesh = plsc.VectorSubcoreMesh(core_axis_name="c", subcore_axis_name="s")

    @functools.partial(
        pl.kernel,
        out_type=jax.ShapeDtypeStruct((npad, NC * DH), jnp.float32),
        mesh=mesh,
        scratch_types=[
            pltpu.VMEM((2, NSUB, K), jnp.int32),
            pltpu.VMEM((2, NSUB, K), jnp.int32),
            pltpu.VMEM((2, NSUB, K), jnp.int32),
            pltpu.VMEM((2, K, DH), jnp.float32),
            pltpu.VMEM((2, K, 2 * DH), jnp.float32),
            pltpu.VMEM((K, 2 * DH), jnp.float32),
            pltpu.VMEM_SHARED((npad, 2 * DH), jnp.float32),
            pltpu.VMEM((DH,), jnp.float32),
            pltpu.SemaphoreType.DMA,
            pltpu.SemaphoreType.DMA,
            pltpu.SemaphoreType.DMA,
            pltpu.SemaphoreType.DMA,
            pltpu.SemaphoreType.DMA,
            pltpu.SemaphoreType.DMA,
        ],
        compiler_params=pltpu.CompilerParams(use_tc_tiling_on_sc=False),
    )
    def sc_edge(src_hbm, dst_hbm, ai_hbm, jh_hbm, bias_hbm, out_hbm,
                srcv, srcg, dstg, abuf, jhbuf, vpbuf, numden,
                biasv, sema0, sema1, semj0, semj1, semi, semv):
        c = lax.axis_index("c")
        s = lax.axis_index("s")
        cn = c * n
        sem_a = (sema0, sema1)
        sem_j = (semj0, semj1)

        ebase0 = s * EC
        for i0 in range(NSUB):
            pltpu.async_copy(src_hbm.at[pl.ds(ebase0 + i0 * K, K)],
                             srcv.at[0, i0], semi)
            pltpu.async_copy(dst_hbm.at[pl.ds(ebase0 + i0 * K, K)],
                             dstg.at[0, i0], semi)

        zero = jnp.zeros((L,), jnp.float32)
        zsrc = jhbuf.at[0, pl.ds(0, RCH)]

        def zrow(r, _):
            for g in range(2 * DH // L):
                jhbuf[0, r, pl.ds(g * L, L)] = zero
            return 0

        lax.fori_loop(0, RCH, zrow, 0)

        def zfire(jr, _):
            pltpu.async_copy(zsrc, numden.at[pl.ds(s * RPT + jr * RCH, RCH)],
                             semv)
            return 0

        lax.fori_loop(0, RPT // RCH, zfire, 0)

        def zwait(jr, _):
            pltpu.make_async_copy(
                zsrc, numden.at[pl.ds(s * RPT + jr * RCH, RCH)], semv).wait()
            return 0

        lax.fori_loop(0, RPT // RCH, zwait, 0)
        plsc.subcore_barrier()

        ebase = s * EC

        def fire_idx(sc_i, ib):
            base = ebase + sc_i * SCH
            for i in range(NSUB):
                pltpu.async_copy(src_hbm.at[pl.ds(base + i * K, K)],
                                 srcv.at[ib, i], semi)
                pltpu.async_copy(dst_hbm.at[pl.ds(base + i * K, K)],
                                 dstg.at[ib, i], semi)

        def wait_idx(sc_i, ib):
            base = ebase + sc_i * SCH
            for i in range(NSUB):
                pltpu.make_async_copy(src_hbm.at[pl.ds(base + i * K, K)],
                                      srcv.at[ib, i], semi).wait()
                pltpu.make_async_copy(dst_hbm.at[pl.ds(base + i * K, K)],
                                      dstg.at[ib, i], semi).wait()

        H = K // 2

        def fire(ib, k, buf):
            for q in range(2):
                pltpu.async_copy(ai_hbm.at[srcg.at[ib, k, pl.ds(q * H, H)]],
                                 abuf.at[buf, pl.ds(q * H, H)], sem_a[buf])
                pltpu.async_copy(jh_hbm.at[dstg.at[ib, k, pl.ds(q * H, H)]],
                                 jhbuf.at[buf, pl.ds(q * H, H)], sem_j[buf])

        def wait_gathers(ib, k, buf):
            for q in range(2):
                pltpu.make_async_copy(
                    ai_hbm.at[srcg.at[ib, k, pl.ds(q * H, H)]],
                    abuf.at[buf, pl.ds(q * H, H)], sem_a[buf]).wait()
                pltpu.make_async_copy(
                    jh_hbm.at[dstg.at[ib, k, pl.ds(q * H, H)]],
                    jhbuf.at[buf, pl.ds(q * H, H)], sem_j[buf]).wait()

        def wait_scatter():
            pltpu.make_async_copy(vpbuf, numden.at[srcv.at[0, 0]],
                                  semv).wait()

        def compute(buf):
            @plsc.parallel_loop(0, K, 1, unroll=8)
            def edge(ei):
                for g in range(DH // L):
                    a = abuf[buf, ei, pl.ds(g * L, L)]
                    b = jhbuf[buf, ei, pl.ds(g * L, L)]
                    hh = jhbuf[buf, ei, pl.ds(DH + g * L, L)]
                    lg = a + b
                    lg = jnp.maximum(lg, 0.2 * lg)
                    p = jnp.exp(lg)
                    vpbuf[ei, pl.ds(g * L, L)] = p * hh
                    vpbuf[ei, pl.ds(DH + g * L, L)] = p

        def fire_scatter(ib, k):
            pltpu.async_copy(vpbuf, numden.at[srcv.at[ib, k]], semv, add=True)

        def addcn(ib):
            def body(i, _):
                for q in range(K // L):
                    sl = pl.ds(q * L, L)
                    srcg[ib, i, sl] = srcv[ib, i, sl] + cn
                    dstg[ib, i, sl] = dstg[ib, i, sl] + cn
                return 0

            lax.fori_loop(0, NSUB, body, 0)

        def superchunk(sc_i, _):
            ib = lax.rem(sc_i, 2)
            wait_idx(sc_i, ib)
            addcn(ib)

            @pl.when(sc_i + 1 < NSUPER)
            def _():
                fire_idx(sc_i + 1, 1 - ib)

            fire(ib, 0, 0)

            def pipe(k2, _):
                k = 2 * k2
                fire(ib, k + 1, 1)
                wait_gathers(ib, k, 0)

                @pl.when((k2 > 0) | (sc_i > 0))
                def _():
                    wait_scatter()

                compute(0)
                fire_scatter(ib, k)

                @pl.when(k + 2 < NSUB)
                def _():
                    fire(ib, k + 2, 0)

                wait_gathers(ib, k + 1, 1)
                wait_scatter()
                compute(1)
                fire_scatter(ib, k + 1)
                return 0

            lax.fori_loop(0, NSUB // 2, pipe, 0)
            return 0

        lax.fori_loop(0, NSUPER, superchunk, 0)
        wait_scatter()
        plsc.subcore_barrier()

        pltpu.sync_copy(bias_hbm.at[pl.ds(c * DH, DH)], biasv)
        NCHE = RPT // RCH

        def nd_dst(b):
            return jhbuf.at[b, pl.ds(0, RCH)]

        def o_src(b):
            return abuf.at[b, pl.ds(0, RCH)]

        def fire_nd(jr, b):
            pltpu.async_copy(numden.at[pl.ds(s * RPT + jr * RCH, RCH)],
                             nd_dst(b), sem_a[b])

        def wait_nd(jr, b):
            pltpu.make_async_copy(numden.at[pl.ds(s * RPT + jr * RCH, RCH)],
                                  nd_dst(b), sem_a[b]).wait()

        def fire_out(jr, b):
            pltpu.async_copy(o_src(b),
                             out_hbm.at[pl.ds(s * RPT + jr * RCH, RCH),
                                        pl.ds(c * DH, DH)], sem_j[b])

        def wait_out(jr, b):
            pltpu.make_async_copy(o_src(b),
                                  out_hbm.at[pl.ds(s * RPT + jr * RCH, RCH),
                                             pl.ds(c * DH, DH)],
                                  sem_j[b]).wait()

        def ecompute(b):
            @plsc.parallel_loop(0, RCH, 1, unroll=4)
            def row(r):
                for g in range(DH // L):
                    nm = jhbuf[b, r, pl.ds(g * L, L)]
                    dn = jhbuf[b, r, pl.ds(DH + g * L, L)]
                    bv = biasv[pl.ds(g * L, L)]
                    abuf[b, r, pl.ds(g * L, L)] = nm / (dn + 1e-16) + bv

        fire_nd(0, 0)

        def echunk(jr2, _):
            jr = 2 * jr2

            @pl.when(jr + 1 < NCHE)
            def _():
                fire_nd(jr + 1, 1)

            wait_nd(jr, 0)

            @pl.when(jr2 > 0)
            def _():
                wait_out(jr, 0)

            ecompute(0)
            fire_out(jr, 0)

            @pl.when(jr + 1 < NCHE)
            def _():
                @pl.when(jr + 2 < NCHE)
                def _():
                    fire_nd(jr + 2, 0)

                wait_nd(jr + 1, 1)

                @pl.when(jr2 > 0)
                def _():
                    wait_out(jr + 1, 1)

                ecompute(1)
                fire_out(jr + 1, 1)

            return 0

        lax.fori_loop(0, (NCHE + 1) // 2, echunk, 0)
        wait_out(0, 0)

        @pl.when(NCHE > 1)
        def _():
            wait_out(0, 1)

    return sc_edge


def kernel(nodes_ft, adj_list, weight, bias, att_layer_1, att_layer_2):
    n, d = nodes_ft.shape
    e = adj_list.shape[1]
    npad = ((n + 127) // 128) * 128
    ai2, jh = _tc_tables(nodes_ft, weight, att_layer_1, att_layer_2)
    sc_edge = _make_sc_edge_kernel(n, e, npad)
    out = sc_edge(
        adj_list[0],
        adj_list[1],
        ai2.reshape(NC * n, DH),
        jh.reshape(NC * n, 2 * DH),
        bias,
    )
    return out[:n]
